# baseline (speedup 1.0000x reference)
import functools

import jax
import jax.numpy as jnp
from jax import lax
from jax.experimental import pallas as pl
from jax.experimental.pallas import tpu as pltpu
from jax.experimental.pallas import tpu_sc as plsc

_VOCAB = 1000000
_D = 64
_B = 16384
_F = 26
_MAX_POS = 10

_NC = 2
_NS = 16
_NW = _NC * _NS
_BW = _B // _NW
_SUB = 128
_NSUB = _BW // _SUB
_NS_TOT = _F * _NSUB
_QD = 4

_GDN = lax.GatherDimensionNumbers(
    offset_dims=(), collapsed_slice_dims=(0,), start_index_map=(0,))

_mesh = plsc.VectorSubcoreMesh(
    core_axis_name="c", subcore_axis_name="s", num_cores=_NC, num_subcores=_NS
)


@functools.partial(
    pl.kernel,
    out_type=jax.ShapeDtypeStruct((_F, _D, _B), jnp.float32),
    mesh=_mesh,
    compiler_params=pltpu.CompilerParams(
        use_tc_tiling_on_sc=False, needs_layout_passes=False),
    scratch_types=[
        pltpu.VMEM((_F, _BW), jnp.int32),
        pltpu.VMEM((_F, _BW), jnp.int32),
        pltpu.VMEM((_QD, _SUB), jnp.int32),
        pltpu.VMEM((_QD * _SUB, _D), jnp.float32),
        pltpu.VMEM((_D, _SUB), jnp.float32),
        pltpu.VMEM((_MAX_POS, _D), jnp.float32),
        pltpu.VMEM((_D, 16), jnp.float32),
        pltpu.SemaphoreType.DMA,
        pltpu.SemaphoreType.DMA,
        pltpu.SemaphoreType.DMA,
        pltpu.SemaphoreType.DMA,
    ],
)
def _sc_embed(idxt_hbm, fvt_hbm, word2_hbm, pos_hbm, out_hbm,
              idxs_v, fvs_v, idxh_v, rows_v, colbuf_v, pos_v, post_v,
              semg0, semg1, semg2, semg3):
    wid = lax.axis_index("s") * _NC + lax.axis_index("c")
    b0 = pl.multiple_of(wid * _BW, _BW)
    pltpu.sync_copy(pos_hbm, pos_v)
    pltpu.sync_copy(idxt_hbm.at[:, pl.ds(b0, _BW)], idxs_v)
    pltpu.sync_copy(fvt_hbm.at[:, pl.ds(b0, _BW)], fvs_v)
    iota16 = jax.lax.broadcasted_iota(jnp.int32, (16,), 0)
    lanei = jnp.minimum(iota16, _MAX_POS - 1)
    for j in range(_D):
        post_v[j, pl.ds(0, 16)] = plsc.load_gather(
            pos_v, [lanei, jnp.full((16,), j, jnp.int32)])
    semg = (semg0, semg1, semg2, semg3)

    def fire(s, slot):
        f = s // _NSUB
        bb = s % _NSUB
        for t in range(_SUB // 16):
            idxh_v[slot, pl.ds(t * 16, 16)] = (
                idxs_v[f, pl.ds(bb * _SUB + t * 16, 16)] << 1)
        pltpu.make_async_copy(
            word2_hbm.at[idxh_v.at[slot]],
            rows_v.at[pl.ds(slot * _SUB, _SUB)],
            semg[slot],
        ).start()

    def add_pass(s, slot):
        f = s // _NSUB
        bb = s % _NSUB
        for jblk in range(_D // 8):
            pjs = [post_v[jblk * 8 + q, pl.ds(0, 16)] for q in range(8)]

            @plsc.parallel_loop(0, _SUB // 16, unroll=2)
            def _(g):
                fg = fvs_v[f, pl.ds(bb * _SUB + g * 16, 16)]
                bvec = iota16 + (slot * _SUB + g * 16)
                for q in range(8):
                    j = jblk * 8 + q
                    w = plsc.load_gather(
                        rows_v, [bvec, jnp.full((16,), j, jnp.int32)])
                    p = lax.gather(
                        pjs[q], fg[:, None], _GDN, slice_sizes=(1,),
                        mode=lax.GatherScatterMode.PROMISE_IN_BOUNDS)
                    colbuf_v[j, pl.ds(g * 16, 16)] = w + p

        bsub = pl.multiple_of(b0 + bb * _SUB, _SUB)
        pltpu.sync_copy(colbuf_v, out_hbm.at[f, :, pl.ds(bsub, _SUB)])

    for l in range(_QD):
        fire(l, l)

    def quad_body(q, carry):
        s = q * _QD
        for l in range(_QD):
            pltpu.make_async_copy(
                word2_hbm.at[idxh_v.at[l]],
                rows_v.at[pl.ds(l * _SUB, _SUB)], semg[l]).wait()
            add_pass(s + l, l)

            @pl.when(q < _NS_TOT // _QD - 1)
            def _(_l=l):
                fire(s + _QD + _l, _l)
        return carry

    lax.fori_loop(0, _NS_TOT // _QD, quad_body, 0)


def kernel(feature_idx, feature_val, word_table, pos_table):
    idxt = feature_idx.astype(jnp.int32).T
    fvt = feature_val.astype(jnp.int32).reshape(_B, _F).T
    word2 = jnp.pad(word_table, ((0, 0), (0, _D))).reshape(2 * _VOCAB, _D)
    out = _sc_embed(idxt, fvt, word2, pos_table)
    return jnp.transpose(out, (2, 0, 1))


# speedup vs baseline: 1.1408x; 1.1408x over previous
import functools

import jax
import jax.numpy as jnp
from jax import lax
from jax.experimental import pallas as pl
from jax.experimental.pallas import tpu as pltpu
from jax.experimental.pallas import tpu_sc as plsc

_VOCAB = 1000000
_D = 64
_B = 16384
_F = 26
_MAX_POS = 10

_NC = 2
_NS = 16
_NW = _NC * _NS
_BW = _B // _NW
_SUB = 128
_NSUB = _BW // _SUB
_NS_TOT = _F * _NSUB
_NPAIR = _NS_TOT // 2

_GDN = lax.GatherDimensionNumbers(
    offset_dims=(), collapsed_slice_dims=(0,), start_index_map=(0,))

_mesh = plsc.VectorSubcoreMesh(
    core_axis_name="c", subcore_axis_name="s", num_cores=_NC, num_subcores=_NS
)


@functools.partial(
    pl.kernel,
    out_type=jax.ShapeDtypeStruct((_F, _D // 8, _B // 128, 8, 128), jnp.float32),
    mesh=_mesh,
    compiler_params=pltpu.CompilerParams(
        use_tc_tiling_on_sc=False, needs_layout_passes=False),
    scratch_types=[
        pltpu.VMEM((_F, _BW), jnp.int32),
        pltpu.VMEM((_F, _BW), jnp.int32),
        pltpu.VMEM((2, _SUB), jnp.int32),
        pltpu.VMEM((2 * _SUB, _D), jnp.float32),
        pltpu.VMEM((2 * _D // 8, 8, _SUB), jnp.float32),
        pltpu.VMEM((_MAX_POS, _D), jnp.float32),
        pltpu.VMEM((_D, 16), jnp.float32),
        pltpu.SemaphoreType.DMA,
        pltpu.SemaphoreType.DMA,
        pltpu.SemaphoreType.DMA,
        pltpu.SemaphoreType.DMA,
    ],
)
def _sc_embed(idxt_hbm, fvt_hbm, word2_hbm, pos_hbm, out_hbm,
              idxs_v, fvs_v, idxh_v, rows_v, colbuf_v, pos_v, post_v,
              semg0, semg1, semo0, semo1):
    wid = lax.axis_index("s") * _NC + lax.axis_index("c")
    b0 = pl.multiple_of(wid * _BW, _BW)
    pltpu.sync_copy(pos_hbm, pos_v)
    pltpu.sync_copy(idxt_hbm.at[:, pl.ds(b0, _BW)], idxs_v)
    pltpu.sync_copy(fvt_hbm.at[:, pl.ds(b0, _BW)], fvs_v)
    iota16 = jax.lax.broadcasted_iota(jnp.int32, (16,), 0)
    lanei = jnp.minimum(iota16, _MAX_POS - 1)
    for j in range(_D):
        post_v[j, pl.ds(0, 16)] = plsc.load_gather(
            pos_v, [lanei, jnp.full((16,), j, jnp.int32)])
    semg = (semg0, semg1)
    semo = (semo0, semo1)

    def fire(s, par):
        f = s // _NSUB
        bb = s % _NSUB
        for t in range(_SUB // 16):
            idxh_v[par, pl.ds(t * 16, 16)] = (
                idxs_v[f, pl.ds(bb * _SUB + t * 16, 16)] << 1)
        pltpu.make_async_copy(
            word2_hbm.at[idxh_v.at[par]],
            rows_v.at[pl.ds(par * _SUB, _SUB)],
            semg[par],
        ).start()

    def out_copy_desc(s, par):
        f = s // _NSUB
        bt = (b0 + (s % _NSUB) * _SUB) // _SUB
        return pltpu.make_async_copy(
            colbuf_v.at[pl.ds(par * (_D // 8), _D // 8)],
            out_hbm.at[f, :, bt, :, :],
            semo[par],
        )

    def add_pass(s, par):
        f = s // _NSUB
        bb = s % _NSUB
        for jblk in range(_D // 8):
            pjs = [post_v[jblk * 8 + q, pl.ds(0, 16)] for q in range(8)]

            @plsc.parallel_loop(0, _SUB // 16, unroll=2)
            def _(g):
                fg = fvs_v[f, pl.ds(bb * _SUB + g * 16, 16)]
                bvec = iota16 + (par * _SUB + g * 16)
                for q in range(8):
                    j = jblk * 8 + q
                    w = plsc.load_gather(
                        rows_v, [bvec, jnp.full((16,), j, jnp.int32)])
                    p = lax.gather(
                        pjs[q], fg[:, None], _GDN, slice_sizes=(1,),
                        mode=lax.GatherScatterMode.PROMISE_IN_BOUNDS)
                    colbuf_v[par * (_D // 8) + j // 8, j % 8,
                             pl.ds(g * 16, 16)] = w + p

    fire(0, 0)

    def pair_body(ss, carry):
        s0 = ss * 2
        s1 = s0 + 1
        fire(s1, 1)

        @pl.when(ss > 0)
        def _():
            out_copy_desc(s0 - 2, 0).wait()
            out_copy_desc(s1 - 2, 1).wait()

        pltpu.make_async_copy(
            word2_hbm.at[idxh_v.at[0]],
            rows_v.at[pl.ds(0, _SUB)], semg[0]).wait()
        add_pass(s0, 0)
        out_copy_desc(s0, 0).start()

        @pl.when(ss < _NPAIR - 1)
        def _():
            fire(s0 + 2, 0)

        pltpu.make_async_copy(
            word2_hbm.at[idxh_v.at[1]],
            rows_v.at[pl.ds(_SUB, _SUB)], semg[1]).wait()
        add_pass(s1, 1)
        out_copy_desc(s1, 1).start()
        return carry

    lax.fori_loop(0, _NPAIR, pair_body, 0)
    out_copy_desc(_NS_TOT - 2, 0).wait()
    out_copy_desc(_NS_TOT - 1, 1).wait()


def kernel(feature_idx, feature_val, word_table, pos_table):
    idxt = feature_idx.astype(jnp.int32).T
    fvt = feature_val.astype(jnp.int32).reshape(_B, _F).T
    word2 = jnp.pad(word_table, ((0, 0), (0, _D))).reshape(2 * _VOCAB, _D)
    out5 = _sc_embed(idxt, fvt, word2, pos_table)
    return jnp.transpose(out5, (2, 4, 0, 1, 3)).reshape(_B, _F, _D)
